# Initial kernel scaffold; baseline (speedup 1.0000x reference)
import functools

import jax
import jax.numpy as jnp
from jax import lax
from jax.experimental import pallas as pl
from jax.experimental.pallas import tpu as pltpu
from jax.experimental.pallas import tpu_sc as plsc

N_ROWS = 100000
FEA = 92
CHUNK = 125
N_CHUNKS = N_ROWS // CHUNK

_info = plsc.get_sparse_core_info()
_NC, _NS = _info.num_cores, _info.num_subcores
N_WORKERS = _NC * _NS
CHUNKS_PER_W = N_CHUNKS // N_WORKERS


def _sc_gather(x2d, atom_fea):
    mesh = plsc.VectorSubcoreMesh(core_axis_name="c", subcore_axis_name="s")

    @functools.partial(
        pl.kernel,
        out_type=jax.ShapeDtypeStruct((N_CHUNKS, CHUNK, FEA), jnp.float32),
        mesh=mesh,
        scratch_types=[
            pltpu.VMEM((CHUNK,), jnp.int32),
            pltpu.VMEM((CHUNK, FEA), jnp.float32),
            pltpu.SemaphoreType.DMA,
        ],
    )
    def k(idx_hbm, table_hbm, out_hbm, idx_v, rows_v, sem):
        wid = lax.axis_index("s") * _NC + lax.axis_index("c")
        base = wid * CHUNKS_PER_W

        def body(j, carry):
            c = base + j
            pltpu.sync_copy(idx_hbm.at[c], idx_v)
            pltpu.async_copy(table_hbm.at[idx_v], rows_v, sem).wait()
            pltpu.sync_copy(rows_v, out_hbm.at[c])
            return carry

        lax.fori_loop(0, CHUNKS_PER_W, body, 0)

    return k(x2d, atom_fea)


def kernel(x, atom_fea):
    x2d = x.reshape(N_CHUNKS, CHUNK)
    out = _sc_gather(x2d, atom_fea)
    return out.reshape(N_ROWS, FEA)


# speedup vs baseline: 1.2575x; 1.2575x over previous
import functools

import jax
import jax.numpy as jnp
from jax import lax
from jax.experimental import pallas as pl
from jax.experimental.pallas import tpu as pltpu
from jax.experimental.pallas import tpu_sc as plsc

N_ROWS = 100000
FEA = 92
W = 96
GROUPS = 25
CHUNK = GROUPS * 16
N_CHUNKS = N_ROWS // CHUNK
CHUNK_WORDS = CHUNK * FEA
OFFS = (0, 16, 32, 48, 64, 76)

_info = plsc.get_sparse_core_info()
_NC, _NS = _info.num_cores, _info.num_subcores
N_WORKERS = _NC * _NS


def _sc_lookup(x2d, table_flat):
    mesh = plsc.VectorSubcoreMesh(core_axis_name="c", subcore_axis_name="s")

    @functools.partial(
        pl.kernel,
        out_type=jax.ShapeDtypeStruct((N_CHUNKS, CHUNK_WORDS), jnp.float32),
        mesh=mesh,
        scratch_types=[
            pltpu.VMEM((102 * W,), jnp.float32),
            pltpu.VMEM((CHUNK,), jnp.int32),
            pltpu.VMEM((CHUNK_WORDS,), jnp.float32),
            pltpu.VMEM((CHUNK_WORDS,), jnp.float32),
            pltpu.SemaphoreType.DMA,
            pltpu.SemaphoreType.DMA,
        ],
        compiler_params=pltpu.CompilerParams(use_tc_tiling_on_sc=False),
    )
    def k(idx_hbm, table_hbm, out_hbm, tab_v, idx_v, pack0, pack1, sem0, sem1):
        wid = lax.axis_index("s") * _NC + lax.axis_index("c")
        pltpu.sync_copy(table_hbm, tab_v)

        n_extra = N_CHUNKS - 7 * N_WORKERS
        nj = 7 + jnp.where(wid < n_extra, 1, 0)

        def fill(c, pack):
            pltpu.sync_copy(idx_hbm.at[c], idx_v)

            def group(g, carry):
                idx16 = idx_v[pl.ds(g * 16, 16)]
                base = g * 16 * FEA
                for l in range(16):
                    src = idx16[l] * W
                    dst = base + l * FEA
                    for off in OFFS:
                        pack[pl.ds(dst + off, 16)] = tab_v[pl.ds(src + off, 16)]
                return carry

            lax.fori_loop(0, GROUPS, group, 0)

        def body(j, carry):
            c = wid + N_WORKERS * j

            @pl.when(j % 2 == 0)
            def _():
                @pl.when(j >= 2)
                def _():
                    pltpu.make_async_copy(pack0, out_hbm.at[c], sem0).wait()
                fill(c, pack0)
                pltpu.async_copy(pack0, out_hbm.at[c], sem0)

            @pl.when(j % 2 == 1)
            def _():
                @pl.when(j >= 2)
                def _():
                    pltpu.make_async_copy(pack1, out_hbm.at[c], sem1).wait()
                fill(c, pack1)
                pltpu.async_copy(pack1, out_hbm.at[c], sem1)

            return carry

        lax.fori_loop(0, nj, body, 0)

        @pl.when(nj >= 1)
        def _():
            c0 = wid
            pltpu.make_async_copy(pack0, out_hbm.at[c0], sem0).wait()

        @pl.when(nj >= 2)
        def _():
            c0 = wid
            pltpu.make_async_copy(pack1, out_hbm.at[c0], sem1).wait()

    return k(x2d, table_flat)


def kernel(x, atom_fea):
    x2d = x.reshape(N_CHUNKS, CHUNK)
    table = jnp.pad(atom_fea, ((0, 0), (0, W - FEA))).reshape(-1)
    out = _sc_lookup(x2d, table)
    return out.reshape(N_ROWS, FEA)


# speedup vs baseline: 1.2591x; 1.0013x over previous
import functools

import jax
import jax.numpy as jnp
from jax import lax
from jax.experimental import pallas as pl
from jax.experimental.pallas import tpu as pltpu
from jax.experimental.pallas import tpu_sc as plsc

N_ROWS = 100000
FEA = 92
W = 96
GROUPS = 25
CHUNK = GROUPS * 16
N_CHUNKS = N_ROWS // CHUNK
CHUNK_WORDS = CHUNK * FEA
OFFS = (0, 16, 32, 48, 64, 76)

_info = plsc.get_sparse_core_info()
_NC, _NS = _info.num_cores, _info.num_subcores
N_WORKERS = _NC * _NS


def _sc_lookup(x2d, table_flat):
    mesh = plsc.VectorSubcoreMesh(core_axis_name="c", subcore_axis_name="s")

    @functools.partial(
        pl.kernel,
        out_type=jax.ShapeDtypeStruct((N_CHUNKS * CHUNK_WORDS,), jnp.float32),
        mesh=mesh,
        scratch_types=[
            pltpu.VMEM((102 * W,), jnp.float32),
            pltpu.VMEM((CHUNK,), jnp.int32),
            pltpu.VMEM((CHUNK_WORDS,), jnp.float32),
            pltpu.VMEM((CHUNK_WORDS,), jnp.float32),
            pltpu.SemaphoreType.DMA,
            pltpu.SemaphoreType.DMA,
        ],
        compiler_params=pltpu.CompilerParams(use_tc_tiling_on_sc=False),
    )
    def k(idx_hbm, table_hbm, out_hbm, tab_v, idx_v, pack0, pack1, sem0, sem1):
        wid = lax.axis_index("s") * _NC + lax.axis_index("c")
        pltpu.sync_copy(table_hbm, tab_v)

        n_extra = N_CHUNKS - 7 * N_WORKERS
        nj = 7 + jnp.where(wid < n_extra, 1, 0)

        def fill(c, pack):
            pltpu.sync_copy(idx_hbm.at[pl.ds(c * CHUNK, CHUNK)], idx_v)

            def group(g, carry):
                idx16 = idx_v[pl.ds(g * 16, 16)]
                base = g * 16 * FEA
                for l in range(16):
                    src = idx16[l] * W
                    dst = base + l * FEA
                    for off in OFFS:
                        pack[pl.ds(dst + off, 16)] = tab_v[pl.ds(src + off, 16)]
                return carry

            lax.fori_loop(0, GROUPS, group, 0)

        def body(j, carry):
            c = wid + N_WORKERS * j

            dst = out_hbm.at[pl.ds(c * CHUNK_WORDS, CHUNK_WORDS)]

            @pl.when(j % 2 == 0)
            def _():
                @pl.when(j >= 2)
                def _():
                    pltpu.make_async_copy(pack0, dst, sem0).wait()
                fill(c, pack0)
                pltpu.async_copy(pack0, dst, sem0)

            @pl.when(j % 2 == 1)
            def _():
                @pl.when(j >= 2)
                def _():
                    pltpu.make_async_copy(pack1, dst, sem1).wait()
                fill(c, pack1)
                pltpu.async_copy(pack1, dst, sem1)

            return carry

        lax.fori_loop(0, nj, body, 0)

        dst0 = out_hbm.at[pl.ds(wid * CHUNK_WORDS, CHUNK_WORDS)]

        @pl.when(nj >= 1)
        def _():
            pltpu.make_async_copy(pack0, dst0, sem0).wait()

        @pl.when(nj >= 2)
        def _():
            pltpu.make_async_copy(pack1, dst0, sem1).wait()

    return k(x2d, table_flat)


def kernel(x, atom_fea):
    table = jnp.pad(atom_fea, ((0, 0), (0, W - FEA))).reshape(-1)
    out = _sc_lookup(x, table)
    return out.reshape(N_ROWS, FEA)


# speedup vs baseline: 1.4343x; 1.1391x over previous
import functools

import jax
import jax.numpy as jnp
from jax import lax
from jax.experimental import pallas as pl
from jax.experimental.pallas import tpu as pltpu
from jax.experimental.pallas import tpu_sc as plsc

N_ROWS = 100000
FEA = 92
W = 96
CHUNK = 400
N_CHUNKS = N_ROWS // CHUNK
SUBS = (0, 128, 256, 384)
SUB_SIZES = (128, 128, 128, 16)

_info = plsc.get_sparse_core_info()
_NC, _NS = _info.num_cores, _info.num_subcores
N_WORKERS = _NC * _NS


def _sc_lookup(x, table_pad):
    mesh = plsc.VectorSubcoreMesh(core_axis_name="c", subcore_axis_name="s")

    @functools.partial(
        pl.kernel,
        out_type=jax.ShapeDtypeStruct((N_ROWS, W), jnp.float32),
        mesh=mesh,
        scratch_types=[
            pltpu.VMEM((CHUNK,), jnp.int32),
            pltpu.VMEM((CHUNK, W), jnp.float32),
            pltpu.VMEM((CHUNK, W), jnp.float32),
            pltpu.SemaphoreType.DMA,
            pltpu.SemaphoreType.DMA,
            pltpu.SemaphoreType.DMA,
        ],
        compiler_params=pltpu.CompilerParams(use_tc_tiling_on_sc=False),
    )
    def k(idx_hbm, table_hbm, out_hbm, idx_v, rows0, rows1, gsem, sem0, sem1):
        wid = lax.axis_index("s") * _NC + lax.axis_index("c")

        n_extra = N_CHUNKS - 7 * N_WORKERS
        nj = 7 + jnp.where(wid < n_extra, 1, 0)

        def gather(c, rows):
            pltpu.sync_copy(idx_hbm.at[pl.ds(c * CHUNK, CHUNK)], idx_v)
            descs = []
            for s, n in zip(SUBS, SUB_SIZES):
                descs.append(
                    pltpu.async_copy(
                        table_hbm.at[idx_v.at[pl.ds(s, n)]],
                        rows.at[pl.ds(s, n), :],
                        gsem,
                    )
                )
            for d in descs:
                d.wait()

        def body(j, carry):
            c = wid + N_WORKERS * j
            dst = out_hbm.at[pl.ds(c * CHUNK, CHUNK), :]

            @pl.when(j % 2 == 0)
            def _():
                @pl.when(j >= 2)
                def _():
                    pltpu.make_async_copy(rows0, dst, sem0).wait()
                gather(c, rows0)
                pltpu.async_copy(rows0, dst, sem0)

            @pl.when(j % 2 == 1)
            def _():
                @pl.when(j >= 2)
                def _():
                    pltpu.make_async_copy(rows1, dst, sem1).wait()
                gather(c, rows1)
                pltpu.async_copy(rows1, dst, sem1)

            return carry

        lax.fori_loop(0, nj, body, 0)

        dst0 = out_hbm.at[pl.ds(wid * CHUNK, CHUNK), :]

        @pl.when(nj >= 1)
        def _():
            pltpu.make_async_copy(rows0, dst0, sem0).wait()

        @pl.when(nj >= 2)
        def _():
            pltpu.make_async_copy(rows1, dst0, sem1).wait()

    return k(x, table_pad)


def kernel(x, atom_fea):
    table = jnp.pad(atom_fea, ((0, 0), (0, W - FEA)))
    out = _sc_lookup(x, table)
    return out[:, :FEA]


# speedup vs baseline: 2.2502x; 1.5688x over previous
import functools

import jax
import jax.numpy as jnp
from jax import lax
from jax.experimental import pallas as pl
from jax.experimental.pallas import tpu as pltpu
from jax.experimental.pallas import tpu_sc as plsc

N_ROWS = 100000
FEA = 92
W = 96
CHUNK = 400
N_CHUNKS = N_ROWS // CHUNK
SUBS = (0, 128, 256, 384)
SUB_SIZES = (128, 128, 128, 16)

_info = plsc.get_sparse_core_info()
_NC, _NS = _info.num_cores, _info.num_subcores
N_WORKERS = _NC * _NS


def _sc_lookup(x, table_pad):
    mesh = plsc.VectorSubcoreMesh(core_axis_name="c", subcore_axis_name="s")

    @functools.partial(
        pl.kernel,
        out_type=jax.ShapeDtypeStruct((N_ROWS, W), jnp.float32),
        mesh=mesh,
        scratch_types=[
            pltpu.VMEM((CHUNK,), jnp.int32),
            pltpu.VMEM((CHUNK, W), jnp.float32),
            pltpu.VMEM((CHUNK, W), jnp.float32),
            pltpu.VMEM_SHARED((102, W), jnp.float32),
            pltpu.SemaphoreType.DMA,
            pltpu.SemaphoreType.DMA,
            pltpu.SemaphoreType.DMA,
        ],
        compiler_params=pltpu.CompilerParams(use_tc_tiling_on_sc=False),
    )
    def k(idx_hbm, table_hbm, out_hbm, idx_v, rows0, rows1, tab_sh,
          gsem, sem0, sem1):
        wid = lax.axis_index("s") * _NC + lax.axis_index("c")
        sid = lax.axis_index("s")

        @pl.when(sid == 0)
        def _():
            pltpu.sync_copy(table_hbm, tab_sh)

        plsc.subcore_barrier()

        n_extra = N_CHUNKS - 7 * N_WORKERS
        nj = 7 + jnp.where(wid < n_extra, 1, 0)

        def gather(c, rows):
            pltpu.sync_copy(idx_hbm.at[pl.ds(c * CHUNK, CHUNK)], idx_v)
            descs = []
            for s, n in zip(SUBS, SUB_SIZES):
                descs.append(
                    pltpu.async_copy(
                        tab_sh.at[idx_v.at[pl.ds(s, n)]],
                        rows.at[pl.ds(s, n), :],
                        gsem,
                    )
                )
            for d in descs:
                d.wait()

        def body(j, carry):
            c = wid + N_WORKERS * j
            dst = out_hbm.at[pl.ds(c * CHUNK, CHUNK), :]

            @pl.when(j % 2 == 0)
            def _():
                @pl.when(j >= 2)
                def _():
                    pltpu.make_async_copy(rows0, dst, sem0).wait()
                gather(c, rows0)
                pltpu.async_copy(rows0, dst, sem0)

            @pl.when(j % 2 == 1)
            def _():
                @pl.when(j >= 2)
                def _():
                    pltpu.make_async_copy(rows1, dst, sem1).wait()
                gather(c, rows1)
                pltpu.async_copy(rows1, dst, sem1)

            return carry

        lax.fori_loop(0, nj, body, 0)

        dst0 = out_hbm.at[pl.ds(wid * CHUNK, CHUNK), :]

        @pl.when(nj >= 1)
        def _():
            pltpu.make_async_copy(rows0, dst0, sem0).wait()

        @pl.when(nj >= 2)
        def _():
            pltpu.make_async_copy(rows1, dst0, sem1).wait()

    return k(x, table_pad)


def kernel(x, atom_fea):
    table = jnp.pad(atom_fea, ((0, 0), (0, W - FEA)))
    out = _sc_lookup(x, table)
    return out[:, :FEA]


# speedup vs baseline: 2.2783x; 1.0125x over previous
import functools

import jax
import jax.numpy as jnp
from jax import lax
from jax.experimental import pallas as pl
from jax.experimental.pallas import tpu as pltpu
from jax.experimental.pallas import tpu_sc as plsc

N_ROWS = 100000
FEA = 92
W = 96
CHUNK = 400
N_CHUNKS = N_ROWS // CHUNK
SUBS = (0, 128, 256, 384)
SUB_SIZES = (128, 128, 128, 16)

_info = plsc.get_sparse_core_info()
_NC, _NS = _info.num_cores, _info.num_subcores
N_WORKERS = _NC * _NS


def _sc_lookup(x, table_pad):
    mesh = plsc.VectorSubcoreMesh(core_axis_name="c", subcore_axis_name="s")

    @functools.partial(
        pl.kernel,
        out_type=jax.ShapeDtypeStruct((N_ROWS, W), jnp.float32),
        mesh=mesh,
        scratch_types=[
            pltpu.VMEM((CHUNK,), jnp.int32),
            pltpu.VMEM((CHUNK,), jnp.int32),
            pltpu.VMEM((CHUNK, W), jnp.float32),
            pltpu.VMEM((CHUNK, W), jnp.float32),
            pltpu.VMEM_SHARED((102, W), jnp.float32),
            pltpu.SemaphoreType.DMA,
            pltpu.SemaphoreType.DMA,
            pltpu.SemaphoreType.DMA,
            pltpu.SemaphoreType.DMA,
            pltpu.SemaphoreType.DMA,
        ],
        compiler_params=pltpu.CompilerParams(use_tc_tiling_on_sc=False),
    )
    def k(idx_hbm, table_hbm, out_hbm, idx0, idx1, rows0, rows1, tab_sh,
          gsem, isem0, isem1, sem0, sem1):
        wid = lax.axis_index("s") * _NC + lax.axis_index("c")

        pltpu.sync_copy(table_hbm, tab_sh)

        n_extra = N_CHUNKS - 7 * N_WORKERS
        nj = 7 + jnp.where(wid < n_extra, 1, 0)

        def idx_src(j):
            c = wid + N_WORKERS * j
            return idx_hbm.at[pl.ds(c * CHUNK, CHUNK)]

        def prefetch(j, buf, isem):
            pltpu.async_copy(idx_src(j), buf, isem)

        def gather(idx_v, isem, rows):
            pltpu.make_async_copy(idx_src(0), idx_v, isem).wait()
            descs = []
            for s, n in zip(SUBS, SUB_SIZES):
                descs.append(
                    pltpu.async_copy(
                        tab_sh.at[idx_v.at[pl.ds(s, n)]],
                        rows.at[pl.ds(s, n), :],
                        gsem,
                    )
                )
            for d in descs:
                d.wait()

        prefetch(0, idx0, isem0)

        def body(j, carry):
            c = wid + N_WORKERS * j
            dst = out_hbm.at[pl.ds(c * CHUNK, CHUNK), :]

            @pl.when(j + 1 < nj)
            def _():
                @pl.when(j % 2 == 0)
                def _():
                    prefetch(j + 1, idx1, isem1)

                @pl.when(j % 2 == 1)
                def _():
                    prefetch(j + 1, idx0, isem0)

            @pl.when(j % 2 == 0)
            def _():
                @pl.when(j >= 2)
                def _():
                    pltpu.make_async_copy(rows0, dst, sem0).wait()
                gather(idx0, isem0, rows0)
                pltpu.async_copy(rows0, dst, sem0)

            @pl.when(j % 2 == 1)
            def _():
                @pl.when(j >= 2)
                def _():
                    pltpu.make_async_copy(rows1, dst, sem1).wait()
                gather(idx1, isem1, rows1)
                pltpu.async_copy(rows1, dst, sem1)

            return carry

        lax.fori_loop(0, nj, body, 0)

        dst0 = out_hbm.at[pl.ds(wid * CHUNK, CHUNK), :]

        @pl.when(nj >= 1)
        def _():
            pltpu.make_async_copy(rows0, dst0, sem0).wait()

        @pl.when(nj >= 2)
        def _():
            pltpu.make_async_copy(rows1, dst0, sem1).wait()

    return k(x, table_pad)


def kernel(x, atom_fea):
    table = jnp.pad(atom_fea, ((0, 0), (0, W - FEA)))
    out = _sc_lookup(x, table)
    return out[:, :FEA]


# speedup vs baseline: 2.2989x; 1.0091x over previous
import functools

import jax
import jax.numpy as jnp
from jax import lax
from jax.experimental import pallas as pl
from jax.experimental.pallas import tpu as pltpu
from jax.experimental.pallas import tpu_sc as plsc

N_ROWS = 100000
FEA = 92
W = 96
CHUNK = 400
N_CHUNKS = N_ROWS // CHUNK
SUBS = (0, 128, 256, 384)
SUB_SIZES = (128, 128, 128, 16)

_info = plsc.get_sparse_core_info()
_NC, _NS = _info.num_cores, _info.num_subcores
N_WORKERS = _NC * _NS


def _sc_lookup(x, table_pad):
    mesh = plsc.VectorSubcoreMesh(core_axis_name="c", subcore_axis_name="s")

    @functools.partial(
        pl.kernel,
        out_type=jax.ShapeDtypeStruct((N_ROWS, W), jnp.float32),
        mesh=mesh,
        scratch_types=[
            pltpu.VMEM((CHUNK,), jnp.int32),
            pltpu.VMEM((CHUNK,), jnp.int32),
            pltpu.VMEM((CHUNK, W), jnp.float32),
            pltpu.VMEM((CHUNK, W), jnp.float32),
            pltpu.VMEM_SHARED((102, W), jnp.float32),
            pltpu.SemaphoreType.DMA,
            pltpu.SemaphoreType.DMA,
            pltpu.SemaphoreType.DMA,
            pltpu.SemaphoreType.DMA,
            pltpu.SemaphoreType.DMA,
        ],
        compiler_params=pltpu.CompilerParams(use_tc_tiling_on_sc=False),
    )
    def k(idx_hbm, table_hbm, out_hbm, idx0, idx1, rows0, rows1, tab_sh,
          gsem, isem0, isem1, sem0, sem1):
        wid = lax.axis_index("s") * _NC + lax.axis_index("c")

        n_extra = N_CHUNKS - 7 * N_WORKERS
        nj = 7 + jnp.where(wid < n_extra, 1, 0)

        def idx_src(j):
            c = wid + N_WORKERS * j
            return idx_hbm.at[pl.ds(c * CHUNK, CHUNK)]

        def prefetch(j, buf, isem):
            pltpu.async_copy(idx_src(j), buf, isem)

        def gather(idx_v, isem, rows):
            pltpu.make_async_copy(idx_src(0), idx_v, isem).wait()
            descs = []
            for s, n in zip(SUBS, SUB_SIZES):
                descs.append(
                    pltpu.async_copy(
                        tab_sh.at[idx_v.at[pl.ds(s, n)]],
                        rows.at[pl.ds(s, n), :],
                        gsem,
                    )
                )
            for d in descs:
                d.wait()

        prefetch(0, idx0, isem0)

        pltpu.sync_copy(table_hbm, tab_sh)

        def body(j, carry):
            c = wid + N_WORKERS * j
            dst = out_hbm.at[pl.ds(c * CHUNK, CHUNK), :]

            @pl.when(j + 1 < nj)
            def _():
                @pl.when(j % 2 == 0)
                def _():
                    prefetch(j + 1, idx1, isem1)

                @pl.when(j % 2 == 1)
                def _():
                    prefetch(j + 1, idx0, isem0)

            @pl.when(j % 2 == 0)
            def _():
                @pl.when(j >= 2)
                def _():
                    pltpu.make_async_copy(rows0, dst, sem0).wait()
                gather(idx0, isem0, rows0)
                pltpu.async_copy(rows0, dst, sem0)

            @pl.when(j % 2 == 1)
            def _():
                @pl.when(j >= 2)
                def _():
                    pltpu.make_async_copy(rows1, dst, sem1).wait()
                gather(idx1, isem1, rows1)
                pltpu.async_copy(rows1, dst, sem1)

            return carry

        lax.fori_loop(0, nj, body, 0)

        dst0 = out_hbm.at[pl.ds(wid * CHUNK, CHUNK), :]

        @pl.when(nj >= 1)
        def _():
            pltpu.make_async_copy(rows0, dst0, sem0).wait()

        @pl.when(nj >= 2)
        def _():
            pltpu.make_async_copy(rows1, dst0, sem1).wait()

    return k(x, table_pad)


def kernel(x, atom_fea):
    table = jnp.pad(atom_fea, ((0, 0), (0, W - FEA)))
    out = _sc_lookup(x, table)
    return out[:, :FEA]
